# baseline (speedup 1.0000x reference)
import functools

import jax
import jax.numpy as jnp
from jax import lax
from jax.experimental import pallas as pl
from jax.experimental.pallas import tpu as pltpu
from jax.experimental.pallas import tpu_sc as plsc

_DEPTH = 1000
_LANES = 16
_NWORKERS = 32
_CP = 2
_NBUF = 2


def _onehot_sc(idx, zeros_blk, *, n_outer, n_inner):
    per_w = n_outer // _NWORKERS
    n_chunks = per_w // _CP
    n_idx = _CP * n_inner
    n_vregs = (n_idx + _LANES - 1) // _LANES
    mesh = plsc.VectorSubcoreMesh(core_axis_name="c", subcore_axis_name="s")

    @functools.partial(
        pl.kernel,
        out_type=jax.ShapeDtypeStruct((n_outer, n_inner, _DEPTH), jnp.float32),
        mesh=mesh,
        scratch_types=[
            pltpu.VMEM((per_w * n_inner + _LANES,), jnp.int32),
            pltpu.VMEM((_CP, n_inner, _DEPTH), jnp.float32),
            pltpu.VMEM((_CP, n_inner, _DEPTH), jnp.float32),
            pltpu.SemaphoreType.DMA,
            pltpu.SemaphoreType.DMA,
        ],
    )
    def body(idx_hbm, zero_hbm, out_hbm, idx_v, buf0, buf1, sem0, sem1):
        wid = lax.axis_index("s") * 2 + lax.axis_index("c")
        base = wid * per_w
        pltpu.sync_copy(idx_hbm.at[pl.ds(base * n_inner, per_w * n_inner)],
                        idx_v.at[pl.ds(0, per_w * n_inner)])
        pltpu.sync_copy(zero_hbm, buf0)
        pltpu.sync_copy(zero_hbm, buf1)

        bufs = (buf0, buf1)
        sems = (sem0, sem1)
        lane = lax.iota(jnp.int32, _LANES)
        ones = jnp.ones((_LANES,), jnp.float32)
        zeros = jnp.zeros((_LANES,), jnp.float32)

        def spans(g):
            for v in range(n_vregs):
                cv = idx_v[pl.ds(g * n_idx + v * _LANES, _LANES)]
                for l in range(min(_LANES, n_idx - v * _LANES)):
                    n = v * _LANES + l
                    c = cv[l]
                    c0 = pl.multiple_of(c & ~(_LANES - 1), _LANES)
                    yield n // n_inner, n % n_inner, c0, c

        def set_chunk(g, buf):
            for p, r, c0, c in spans(g):
                buf[p, r, pl.ds(c0, _LANES)] = jnp.where(
                    lane + c0 == c, ones, zeros)

        def clear_chunk(g, buf):
            for p, r, c0, _ in spans(g):
                buf[p, r, pl.ds(c0, _LANES)] = zeros

        def out_copy(g, buf, sem):
            dst = out_hbm.at[pl.ds(base + g * _CP, _CP)]
            return pltpu.make_async_copy(buf, dst, sem)

        for b in range(_NBUF):
            set_chunk(b, bufs[b])
            out_copy(b, bufs[b], sems[b]).start()

        def step(i, _):
            g0 = i * _NBUF
            for b in range(_NBUF):
                g = g0 + b
                gp = g - _NBUF
                out_copy(gp, bufs[b], sems[b]).wait()
                clear_chunk(gp, bufs[b])
                set_chunk(g, bufs[b])
                out_copy(g, bufs[b], sems[b]).start()
            return _

        lax.fori_loop(1, n_chunks // _NBUF, step, None)

        for b in range(_NBUF):
            out_copy(n_chunks - _NBUF + b, bufs[b], sems[b]).wait()

    return body(idx, zeros_blk)


def kernel(input_, emb_weight):
    del emb_weight
    n_outer, n_inner = input_.shape
    idx = input_.reshape(n_outer * n_inner).astype(jnp.int32)
    zeros_blk = jnp.zeros((_CP, n_inner, _DEPTH), jnp.float32)
    return _onehot_sc(idx, zeros_blk, n_outer=n_outer, n_inner=n_inner)


# speedup vs baseline: 2.3895x; 2.3895x over previous
import functools

import jax
import jax.numpy as jnp
from jax import lax
from jax.experimental import pallas as pl
from jax.experimental.pallas import tpu as pltpu
from jax.experimental.pallas import tpu_sc as plsc

_DEPTH = 1000
_LANES = 16
_NWORKERS = 32
_TROW = 8
_NBUF = 2
_VGROUP = 16


def _onehot_sc(idx_t, *, n_inner, n_rows):
    n_trows = n_inner * (_DEPTH // _TROW)
    base_n = n_trows // _NWORKERS
    n_extra = n_trows - base_n * _NWORKERS
    n_spans = n_rows // _LANES
    tpj = _DEPTH // _TROW
    mesh = plsc.VectorSubcoreMesh(core_axis_name="c", subcore_axis_name="s")

    @functools.partial(
        pl.kernel,
        out_type=jax.ShapeDtypeStruct((n_inner, _DEPTH, n_rows), jnp.float32),
        mesh=mesh,
        scratch_types=[
            pltpu.VMEM((2 * n_rows,), jnp.int32),
            pltpu.VMEM((_TROW, n_rows), jnp.float32),
            pltpu.VMEM((_TROW, n_rows), jnp.float32),
            pltpu.SemaphoreType.DMA,
            pltpu.SemaphoreType.DMA,
        ],
    )
    def body(idx_hbm, out_hbm, idx_v, buf0, buf1, sem0, sem1):
        wid = lax.axis_index("s") * 2 + lax.axis_index("c")
        start = wid * base_n + jnp.minimum(wid, n_extra)
        n_w = base_n + jnp.where(wid < n_extra, 1, 0)

        j0 = start // tpj
        j1 = jnp.minimum(j0 + 1, n_inner - 1)
        pltpu.sync_copy(idx_hbm.at[pl.ds(j0 * n_rows, n_rows)],
                        idx_v.at[pl.ds(0, n_rows)])
        pltpu.sync_copy(idx_hbm.at[pl.ds(j1 * n_rows, n_rows)],
                        idx_v.at[pl.ds(n_rows, n_rows)])

        bufs = (buf0, buf1)
        sems = (sem0, sem1)

        def fill(buf, jj, tc):
            c_rows = [jnp.full((_LANES,), tc * _TROW + rc, jnp.int32)
                      for rc in range(_TROW)]
            idx_base = jj * n_rows

            def vstep(g, carry):
                v0 = g * _VGROUP
                for v in range(_VGROUP):
                    off = pl.multiple_of((v0 + v) * _LANES, _LANES)
                    cv = idx_v[pl.ds(idx_base + off, _LANES)]
                    for rc in range(_TROW):
                        buf[rc, pl.ds(off, _LANES)] = jnp.where(
                            cv == c_rows[rc], 1.0, 0.0).astype(jnp.float32)
                return carry

            lax.fori_loop(0, n_spans // _VGROUP, vstep, None)

        def out_copy(tr, buf, sem):
            j = tr // tpj
            tc = tr - j * tpj
            dst = out_hbm.at[j, pl.ds(pl.multiple_of(tc * _TROW, _TROW), _TROW), :]
            return pltpu.make_async_copy(buf, dst, sem)

        def step(k, carry):
            b = lax.rem(k, _NBUF)
            tr = start + k
            j = tr // tpj
            jj = j - j0
            tc = tr - j * tpj

            @pl.when(b == 0)
            def _run0():
                @pl.when(k >= _NBUF)
                def _wait():
                    out_copy(tr - _NBUF, buf0, sem0).wait()
                fill(buf0, jj, tc)
                out_copy(tr, buf0, sem0).start()

            @pl.when(b == 1)
            def _run1():
                @pl.when(k >= _NBUF)
                def _wait():
                    out_copy(tr - _NBUF, buf1, sem1).wait()
                fill(buf1, jj, tc)
                out_copy(tr, buf1, sem1).start()

            return carry

        lax.fori_loop(0, n_w, step, None)

        def drain(k):
            @pl.when(k >= 0)
            def _d():
                b = lax.rem(k, _NBUF)

                @pl.when(b == 0)
                def _d0():
                    out_copy(start + k, buf0, sem0).wait()

                @pl.when(b == 1)
                def _d1():
                    out_copy(start + k, buf1, sem1).wait()

        drain(n_w - 2)
        drain(n_w - 1)

    return body(idx_t)


def kernel(input_, emb_weight):
    del emb_weight
    n_rows, n_inner = input_.shape
    idx_t = jnp.transpose(input_).reshape(n_inner * n_rows).astype(jnp.int32)
    t = _onehot_sc(idx_t, n_inner=n_inner, n_rows=n_rows)
    return jnp.transpose(t, (2, 0, 1))
